# baseline (speedup 1.0000x reference)
import functools

import jax
import jax.numpy as jnp
from jax import lax
from jax.experimental import pallas as pl
from jax.experimental.pallas import tpu as pltpu
from jax.experimental.pallas import tpu_sc as plsc

_L = 16
_NS = 16
_NROWS = _NS * _L
_D = 64
_W = 2 * _D


def _sc_body(emb2, idx, out, idx_v, sidx_v, rows_v, srows_v, part_v, comb_v,
             out_v, shared, sem):
    w = lax.axis_index("s")
    base = pl.multiple_of(w * _L, _L)

    pltpu.sync_copy(idx.at[pl.ds(base, _L)], idx_v)
    pltpu.sync_copy(idx.at[pl.ds(_NROWS, _L)], sidx_v)

    vidx = idx_v[...]
    svidx = sidx_v[...]
    sup = lax.shift_right_logical(vidx, 1)
    ssup = lax.shift_right_logical(svidx, 1)
    half = lax.shift_left(jnp.bitwise_and(vidx, 1), 6)
    shalf = lax.shift_left(jnp.bitwise_and(svidx, 1), 6)

    cp_rows = pltpu.async_copy(emb2.at[sup], rows_v, sem)
    cp_src = pltpu.async_copy(emb2.at[ssup], srows_v, sem)
    cp_rows.wait()
    cp_src.wait()

    lanes = lax.iota(jnp.int32, _L)
    zeros = jnp.zeros((_L,), jnp.int32)
    src_chunks = [
        plsc.load_gather(srows_v, [zeros, shalf + (lanes + c * _L)])
        for c in range(_D // _L)
    ]
    acc = jnp.zeros((_L,), jnp.float32)
    for d in range(_D):
        col = plsc.load_gather(rows_v, [lanes, half + d])
        acc = acc + col * src_chunks[d // _L][d % _L]

    lane_r = lanes + base
    neg_mask = lane_r < 200
    ctx_mask = jnp.logical_and(lane_r >= 200, lane_r < 250)
    sig = 1.0 / (1.0 + jnp.exp(acc))
    part_v[0, :] = jnp.where(neg_mask, sig, 0.0)
    part_v[1, :] = jnp.where(ctx_mask, acc, 0.0)
    pltpu.sync_copy(part_v, shared.at[pl.ds(2 * w, 2)])
    plsc.subcore_barrier()

    @pl.when(w == 0)
    def _():
        pltpu.sync_copy(shared, comb_v)
        nacc = jnp.zeros((_L,), jnp.float32)
        pacc = jnp.zeros((_L,), jnp.float32)
        for i in range(_NS):
            nacc = nacc + comb_v[2 * i, :]
            pacc = pacc + comb_v[2 * i + 1, :]
        nsum = jnp.sum(nacc)
        psum = jnp.sum(pacc)
        pos = 1.0 / (1.0 + jnp.exp(-(jnp.zeros((_L,), jnp.float32) + psum)))
        posc = jnp.clip(pos, 1e-7, 1.0 - 1e-7)
        negc = jnp.clip(jnp.zeros((_L,), jnp.float32) + nsum, 1e-7, 1.0 - 1e-7)
        out_v[...] = jnp.where(lanes == 0, posc, negc)
        pltpu.sync_copy(out_v, out)


@jax.jit
def _sc_loss_parts(emb2, idx):
    f = pl.kernel(
        _sc_body,
        out_type=jax.ShapeDtypeStruct((_L,), jnp.float32),
        mesh=plsc.VectorSubcoreMesh(
            core_axis_name="c", subcore_axis_name="s",
            num_cores=1, num_subcores=_NS),
        scratch_types=[
            pltpu.VMEM((_L,), jnp.int32),
            pltpu.VMEM((_L,), jnp.int32),
            pltpu.VMEM((_L, _W), jnp.float32),
            pltpu.VMEM((_L, _W), jnp.float32),
            pltpu.VMEM((2, _L), jnp.float32),
            pltpu.VMEM((2 * _NS, _L), jnp.float32),
            pltpu.VMEM((_L,), jnp.float32),
            pltpu.VMEM_SHARED((2 * _NS, _L), jnp.float32),
            pltpu.SemaphoreType.DMA,
        ],
        compiler_params=pltpu.CompilerParams(needs_layout_passes=False),
    )
    return f(emb2, idx)


def kernel(embedding, source_node, context_nodes, neg_samples):
    emb2 = embedding.reshape(500000, _W)
    idx = jnp.concatenate([
        neg_samples.astype(jnp.int32),
        context_nodes.astype(jnp.int32),
        jnp.zeros((6,), jnp.int32),
        jnp.broadcast_to(source_node.astype(jnp.int32), (_L,)),
    ])
    parts = _sc_loss_parts(emb2, idx)
    return -jnp.log(parts[0]) - parts[1]


# speedup vs baseline: 1.7695x; 1.7695x over previous
import jax
import jax.numpy as jnp
from jax import lax
from jax.experimental import pallas as pl
from jax.experimental.pallas import tpu as pltpu
from jax.experimental.pallas import tpu_sc as plsc

_L = 16
_NS = 16
_D = 64
_LN2 = 0.6931471805599453


def _sc_body(emb, neg, ctx, src, out, negv, ctxv, srcv, rows_v, srows_v,
             part_v, comb_v, out_v, shared, sem):
    w = lax.axis_index("s")
    base = pl.multiple_of(w * _L, _L)

    pltpu.sync_copy(neg, negv)
    pltpu.sync_copy(ctx, ctxv)
    pltpu.sync_copy(src, srcv)

    lanes = lax.iota(jnp.int32, _L)
    g = lanes + base
    n_i = plsc.load_gather(negv, [jnp.minimum(g, 199)])
    c_i = plsc.load_gather(ctxv, [jnp.clip(g - 200, 0, 49)])
    s_i = plsc.load_gather(srcv, [jnp.zeros((_L,), jnp.int32)])
    vidx = jnp.where(g < 200, n_i, jnp.where(g < 250, c_i, s_i))

    copies = []
    for i in range(_L):
        copies.append(pltpu.async_copy(
            emb.at[pl.ds(vidx[i], 1), :], rows_v.at[pl.ds(i, 1), :], sem))
    copies.append(pltpu.async_copy(
        emb.at[pl.ds(s_i[0], 1), :], srows_v, sem))
    for cp in copies:
        cp.wait()

    def _bf16r(x):
        b = plsc.bitcast(x, jnp.int32)
        r = b + 0x7FFF + jnp.bitwise_and(lax.shift_right_logical(b, 16), 1)
        return plsc.bitcast(jnp.bitwise_and(r, jnp.int32(-65536)), jnp.float32)

    acc = jnp.zeros((_L,), jnp.float32)
    src_chunks = [_bf16r(srows_v[0, pl.ds(c * _L, _L)])
                  for c in range(_D // _L)]
    for d in range(_D):
        col = plsc.load_gather(rows_v, [lanes, jnp.full((_L,), d, jnp.int32)])
        acc = acc + _bf16r(col) * src_chunks[d // _L][d % _L]

    sig = 1.0 / (1.0 + jnp.exp(acc))
    part_v[0, :] = jnp.where(g < 200, sig, 0.0)
    part_v[1, :] = jnp.where(jnp.logical_and(g >= 200, g < 250), acc, 0.0)
    pltpu.sync_copy(part_v, shared.at[pl.ds(2 * w, 2)])
    plsc.subcore_barrier()

    @pl.when(w == 0)
    def _():
        pltpu.sync_copy(shared, comb_v)
        nacc = jnp.zeros((_L,), jnp.float32)
        pacc = jnp.zeros((_L,), jnp.float32)
        for i in range(_NS):
            nacc = nacc + comb_v[2 * i, :]
            pacc = pacc + comb_v[2 * i + 1, :]
        nsum = jnp.zeros((_L,), jnp.float32) + jnp.sum(nacc)
        psum = jnp.zeros((_L,), jnp.float32) + jnp.sum(pacc)
        pos = 1.0 / (1.0 + jnp.exp(-psum))
        posc = jnp.clip(pos, 1e-7, 1.0 - 1e-7)
        negc = jnp.clip(nsum, 1e-7, 1.0 - 1e-7)
        bits = plsc.bitcast(posc, jnp.int32)
        e = lax.shift_right_logical(bits, 23) - 127
        m = plsc.bitcast(
            jnp.bitwise_or(jnp.bitwise_and(bits, 0x007FFFFF), 0x3F800000),
            jnp.float32)
        z = (m - 1.0) / (m + 1.0)
        z2 = z * z
        lnm = 2.0 * z * (1.0 + z2 * (1.0 / 3.0 + z2 * (
            0.2 + z2 * (1.0 / 7.0 + z2 * (1.0 / 9.0)))))
        lnp = e.astype(jnp.float32) * _LN2 + lnm
        out_v[...] = -lnp - negc
        pltpu.sync_copy(out_v.at[pl.ds(0, 8)], out)


@jax.jit
def _sc_loss(emb, neg, ctx, src):
    f = pl.kernel(
        _sc_body,
        out_type=jax.ShapeDtypeStruct((8,), jnp.float32),
        mesh=plsc.VectorSubcoreMesh(
            core_axis_name="c", subcore_axis_name="s",
            num_cores=1, num_subcores=_NS),
        scratch_types=[
            pltpu.VMEM((200,), jnp.int32),
            pltpu.VMEM((50,), jnp.int32),
            pltpu.VMEM((1,), jnp.int32),
            pltpu.VMEM((_L, _D), jnp.float32),
            pltpu.VMEM((1, _D), jnp.float32),
            pltpu.VMEM((2, _L), jnp.float32),
            pltpu.VMEM((2 * _NS, _L), jnp.float32),
            pltpu.VMEM((_L,), jnp.float32),
            pltpu.VMEM_SHARED((2 * _NS, _L), jnp.float32),
            pltpu.SemaphoreType.DMA,
        ],
        compiler_params=pltpu.CompilerParams(needs_layout_passes=False),
    )
    return f(emb, neg, ctx, src)


def kernel(embedding, source_node, context_nodes, neg_samples):
    parts = _sc_loss(
        embedding,
        neg_samples.astype(jnp.int32),
        context_nodes.astype(jnp.int32),
        source_node.astype(jnp.int32),
    )
    return parts[0]


# speedup vs baseline: 22.6153x; 12.7809x over previous
import jax
import jax.numpy as jnp
from jax import lax
from jax.experimental import pallas as pl
from jax.experimental.pallas import tpu as pltpu
from jax.experimental.pallas import tpu_sc as plsc

_L = 16
_NS = 16
_D = 64
_NBUF = 4
_LN2 = 0.6931471805599453


def _sc_body(embT, neg, ctx, src, out, negv, ctxv, srcv, blocks_v, rows_v,
             srows_v, part_v, comb_v, out_v, shared, sem):
    w = lax.axis_index("s")
    base = pl.multiple_of(w * _L, _L)

    pltpu.sync_copy(neg, negv)
    pltpu.sync_copy(ctx, ctxv)
    pltpu.sync_copy(src, srcv)

    lanes = lax.iota(jnp.int32, _L)
    g = lanes + base
    n_i = plsc.load_gather(negv, [jnp.minimum(g, 199)])
    c_i = plsc.load_gather(ctxv, [jnp.clip(g - 200, 0, 49)])
    s_i = plsc.load_gather(srcv, [jnp.zeros((_L,), jnp.int32)])
    vidx = jnp.where(g < 200, n_i, jnp.where(g < 250, c_i, s_i))

    nrows = _L + 1
    rs = [vidx[i] for i in range(_L)] + [s_i[0]]

    def issue(k):
        blk = pl.multiple_of(
            lax.shift_left(lax.shift_right_logical(rs[k], 7), 7), 128)
        return pltpu.async_copy(
            embT.at[:, pl.ds(blk, 128)], blocks_v.at[k % _NBUF], sem)

    def drain(k, cp):
        cp.wait()
        lane = jnp.bitwise_and(rs[k], 127)
        lanevec = jnp.zeros((_L,), jnp.int32) + lane
        for c in range(_D // _L):
            chunk = plsc.load_gather(
                blocks_v.at[k % _NBUF], [c * _L + lanes, lanevec])
            if k < _L:
                rows_v[k, pl.ds(c * _L, _L)] = chunk
            else:
                srows_v[0, pl.ds(c * _L, _L)] = chunk

    pend = {}
    for k in range(nrows):
        if k >= _NBUF:
            drain(k - _NBUF, pend.pop(k - _NBUF))
        pend[k] = issue(k)
    for k in range(nrows - _NBUF, nrows):
        drain(k, pend.pop(k))

    def _bf16r(x):
        b = plsc.bitcast(x, jnp.int32)
        r = b + 0x7FFF + jnp.bitwise_and(lax.shift_right_logical(b, 16), 1)
        return plsc.bitcast(jnp.bitwise_and(r, jnp.int32(-65536)), jnp.float32)

    acc = jnp.zeros((_L,), jnp.float32)
    src_chunks = [_bf16r(srows_v[0, pl.ds(c * _L, _L)])
                  for c in range(_D // _L)]
    for d in range(_D):
        col = plsc.load_gather(rows_v, [lanes, jnp.full((_L,), d, jnp.int32)])
        acc = acc + _bf16r(col) * src_chunks[d // _L][d % _L]

    sig = 1.0 / (1.0 + jnp.exp(acc))
    part_v[0, :] = jnp.where(g < 200, sig, 0.0)
    part_v[1, :] = jnp.where(jnp.logical_and(g >= 200, g < 250), acc, 0.0)
    pltpu.sync_copy(part_v, shared.at[pl.ds(2 * w, 2)])
    plsc.subcore_barrier()

    @pl.when(w == 0)
    def _():
        pltpu.sync_copy(shared, comb_v)
        nacc = jnp.zeros((_L,), jnp.float32)
        pacc = jnp.zeros((_L,), jnp.float32)
        for i in range(_NS):
            nacc = nacc + comb_v[2 * i, :]
            pacc = pacc + comb_v[2 * i + 1, :]
        nsum = jnp.zeros((_L,), jnp.float32) + jnp.sum(nacc)
        psum = jnp.zeros((_L,), jnp.float32) + jnp.sum(pacc)
        pos = 1.0 / (1.0 + jnp.exp(-psum))
        posc = jnp.clip(pos, 1e-7, 1.0 - 1e-7)
        negc = jnp.clip(nsum, 1e-7, 1.0 - 1e-7)
        bits = plsc.bitcast(posc, jnp.int32)
        e = lax.shift_right_logical(bits, 23) - 127
        m = plsc.bitcast(
            jnp.bitwise_or(jnp.bitwise_and(bits, 0x007FFFFF), 0x3F800000),
            jnp.float32)
        z = (m - 1.0) / (m + 1.0)
        z2 = z * z
        lnm = 2.0 * z * (1.0 + z2 * (1.0 / 3.0 + z2 * (
            0.2 + z2 * (1.0 / 7.0 + z2 * (1.0 / 9.0)))))
        lnp = e.astype(jnp.float32) * _LN2 + lnm
        out_v[...] = -lnp - negc
        pltpu.sync_copy(out_v.at[pl.ds(0, 8)], out)


@jax.jit
def _sc_loss(embT, neg, ctx, src):
    f = pl.kernel(
        _sc_body,
        out_type=jax.ShapeDtypeStruct((8,), jnp.float32),
        mesh=plsc.VectorSubcoreMesh(
            core_axis_name="c", subcore_axis_name="s",
            num_cores=1, num_subcores=_NS),
        scratch_types=[
            pltpu.VMEM((200,), jnp.int32),
            pltpu.VMEM((50,), jnp.int32),
            pltpu.VMEM((1,), jnp.int32),
            pltpu.VMEM((_NBUF, _D, 128), jnp.float32),
            pltpu.VMEM((_L, _D), jnp.float32),
            pltpu.VMEM((1, _D), jnp.float32),
            pltpu.VMEM((2, _L), jnp.float32),
            pltpu.VMEM((2 * _NS, _L), jnp.float32),
            pltpu.VMEM((_L,), jnp.float32),
            pltpu.VMEM_SHARED((2 * _NS, _L), jnp.float32),
            pltpu.SemaphoreType.DMA,
        ],
        compiler_params=pltpu.CompilerParams(needs_layout_passes=False),
    )
    return f(embT, neg, ctx, src)


def kernel(embedding, source_node, context_nodes, neg_samples):
    parts = _sc_loss(
        embedding.T,
        neg_samples.astype(jnp.int32),
        context_nodes.astype(jnp.int32),
        source_node.astype(jnp.int32),
    )
    return parts[0]
